# baseline (speedup 1.0000x reference)
import functools

import jax
import jax.numpy as jnp
import numpy as np
from jax import lax
from jax.experimental import pallas as pl
from jax.experimental.pallas import tpu as pltpu
from jax.experimental.pallas import tpu_sc as plsc

B = 16
N = 10000
F = 128
K = 64
NWORK = 32
VREGS = N // 16
UNROLL = 25
OUTER = VREGS // UNROLL
HB = 4096


def _score_body(x_ref, w_ref, o_ref):
    w = w_ref[...]
    for bb in range(B):
        o_ref[bb:bb + 1, :] = lax.dot_general(
            w, x_ref[bb],
            dimension_numbers=(((1,), (1,)), ((), ())),
            preferred_element_type=jnp.float32)


def _scores_tc(x3d, w_row):
    blkn = 1024
    grid = (N + blkn - 1) // blkn
    return pl.pallas_call(
        _score_body,
        grid=(grid,),
        in_specs=[
            pl.BlockSpec((B, blkn, F), lambda i: (0, i, 0)),
            pl.BlockSpec((1, F), lambda i: (0, 0)),
        ],
        out_specs=pl.BlockSpec((B, blkn), lambda i: (0, i)),
        out_shape=jax.ShapeDtypeStruct((B, N), jnp.float32),
    )(x3d, w_row)


_LANES = None


def _f2k(f):
    i = lax.bitcast_convert_type(f, jnp.int32)
    return i ^ (lax.shift_right_arithmetic(i, 31) & jnp.int32(0x7FFFFFFF))


def _k2f(k):
    i = k ^ (lax.shift_right_arithmetic(k, 31) & jnp.int32(0x7FFFFFFF))
    return lax.bitcast_convert_type(i, jnp.float32)


def _lanes():
    return lax.iota(jnp.int32, 16)


def _lsum(v):
    lanes = _lanes()
    for step in (8, 4, 2, 1):
        v = v + v.at[lanes ^ step].get(mode="promise_in_bounds")
    return v


def _lmax(v):
    lanes = _lanes()
    for step in (8, 4, 2, 1):
        v = jnp.maximum(v, v.at[lanes ^ step].get(mode="promise_in_bounds"))
    return v


def _lmin(v):
    lanes = _lanes()
    for step in (8, 4, 2, 1):
        v = jnp.minimum(v, v.at[lanes ^ step].get(mode="promise_in_bounds"))
    return v


def _lsuffix(v):
    lanes = _lanes()
    for step in (1, 2, 4, 8):
        g = v.at[jnp.minimum(lanes + step, 15)].get(mode="promise_in_bounds")
        v = v + jnp.where(lanes + step < 16, g, 0)
    return v


def _select_body(scores_hbm, out_hbm, buf, hist, hist2, res_v, scr_i):
    c = lax.axis_index("c")
    s = lax.axis_index("s")
    wid = s * 2 + c
    batch = wid // 2
    role = wid % 2
    _select_one(scores_hbm, out_hbm, buf, hist, hist2, res_v, scr_i,
                batch, role, wid)


def _to_scalar(scr_i, splat):
    del scr_i
    return splat[0]


def _walk(hist_ref, nchunks, jc0, need, scr_i):
    def chunksum(jc):
        return _to_scalar(scr_i, _lsum(hist_ref[pl.ds(jc * 16, 16)]))

    def cond(cr):
        jc, acc, cs = cr
        return acc + cs < need

    def body(cr):
        jc, acc, cs = cr
        return (jc - 1, acc + cs, chunksum(jc - 1))

    jc, acc, _ = lax.while_loop(cond, body, (jc0, jnp.int32(0),
                                             chunksum(jc0)))
    c16 = hist_ref[pl.ds(jc * 16, 16)]
    suf = _lsuffix(c16)
    total_ge = suf + acc
    gem = (total_ge >= need).astype(jnp.int32)
    i_star = _lsum(gem) - 1
    lanes = _lanes()
    tg = _lsum(jnp.where(lanes == i_star, total_ge, 0))
    cv = _lsum(jnp.where(lanes == i_star, c16, 0))
    bstar = jnp.full((16,), jc, jnp.int32) * 16 + i_star
    return bstar, tg - cv, cv


def _select_one(scores_hbm, out_hbm, buf, hist, hist2, res_v, scr_i,
                batch, role, wid):
    pltpu.sync_copy(scores_hbm.at[batch], buf)
    sgn = jnp.where(jnp.full((16,), role, jnp.int32) == 0,
                    jnp.float32(1.0), jnp.float32(-1.0))
    zi = jnp.zeros((16,), jnp.int32)
    ones = jnp.ones((16,), jnp.int32)

    def zh(j, cr):
        base = j * 256
        for u in range(16):
            hist[pl.ds(base + u * 16, 16)] = zi
        return cr

    lax.fori_loop(0, HB // 256, zh, jnp.int32(0))
    for u in range(16):
        hist2[pl.ds(u * 16, 16)] = zi

    def p1(j, vmax):
        base = j * (UNROLL * 16)
        for u in range(UNROLL):
            sl = pl.ds(base + u * 16, 16)
            v = buf[sl] * sgn
            buf[sl] = v
            k = _f2k(v)
            b1 = lax.shift_right_arithmetic(k, 20) + 2048
            plsc.addupdate_scatter(hist, [b1], ones)
            vmax = jnp.maximum(vmax, v)
        return vmax

    vmax = lax.fori_loop(0, OUTER, p1,
                         jnp.full((16,), -3.4028235e38, jnp.float32))
    kmax = _f2k(_lmax(vmax))
    jc0 = _to_scalar(scr_i,
                     (lax.shift_right_arithmetic(kmax, 20) + 2048) // 16)
    b1s, gt1, _ = _walk(hist, HB // 16, jc0, jnp.int32(K), scr_i)
    need2 = jnp.int32(K) - gt1
    need2_s = _to_scalar(scr_i, need2)

    def p2(j, cr):
        base = j * (UNROLL * 16)
        for u in range(UNROLL):
            v = buf[pl.ds(base + u * 16, 16)]
            k = _f2k(v)
            b1 = lax.shift_right_arithmetic(k, 20) + 2048
            b2 = lax.shift_right_arithmetic(k, 12) & jnp.int32(0xFF)
            plsc.addupdate_scatter(hist2, [b2], ones, mask=b1 == b1s)
        return cr

    lax.fori_loop(0, OUTER, p2, jnp.int32(0))
    b2s, gt2, pcnt = _walk(hist2, 16, jnp.int32(15), need2_s, scr_i)
    need3 = need2 - gt2
    need3_s = _to_scalar(scr_i, need3)
    pcnt_s = _to_scalar(scr_i, pcnt)
    pfx = lax.shift_left(b1s - 2048, 8) | b2s

    def low_bits_short():
        def p3(j, vmin):
            base = j * (UNROLL * 16)
            for u in range(UNROLL):
                k = _f2k(buf[pl.ds(base + u * 16, 16)])
                m = lax.shift_right_arithmetic(k, 12) == pfx
                b3 = k & jnp.int32(0xFFF)
                vmin = jnp.minimum(vmin, jnp.where(m, b3, jnp.int32(HB)))
            return vmin

        vmin = lax.fori_loop(0, OUTER, p3, jnp.full((16,), HB, jnp.int32))
        return _lmin(vmin)

    def low_bits_full():
        def zh3(j, cr):
            base = j * 256
            for u in range(16):
                hist[pl.ds(base + u * 16, 16)] = zi
            return cr

        lax.fori_loop(0, HB // 256, zh3, jnp.int32(0))

        def p3(j, vmax3):
            base = j * (UNROLL * 16)
            for u in range(UNROLL):
                k = _f2k(buf[pl.ds(base + u * 16, 16)])
                m = lax.shift_right_arithmetic(k, 12) == pfx
                b3 = k & jnp.int32(0xFFF)
                plsc.addupdate_scatter(hist, [b3], ones, mask=m)
                vmax3 = jnp.maximum(vmax3, jnp.where(m, b3, jnp.int32(-1)))
            return vmax3

        vmax3 = lax.fori_loop(0, OUTER, p3, jnp.full((16,), -1, jnp.int32))
        jc3 = _to_scalar(scr_i, _lmax(vmax3) // 16)
        b3s, _, _ = _walk(hist, HB // 16, jc3, need3_s, scr_i)
        return b3s

    b3min = lax.cond(pcnt_s == need3_s, low_bits_short, low_bits_full)
    kth = lax.shift_left(pfx, 12) | b3min
    t = _k2f(kth)

    def fpass(j, carry):
        sacc, cacc = carry
        base = j * (UNROLL * 16)
        for u in range(UNROLL):
            v = buf[pl.ds(base + u * 16, 16)]
            g = v > t
            sacc = sacc + jnp.where(g, v, jnp.float32(0.0))
            cacc = cacc + jnp.where(g, 1, 0)
        return sacc, cacc

    sacc, cacc = lax.fori_loop(
        0, OUTER, fpass, (jnp.zeros((16,), jnp.float32), zi))
    ssum = _lsum(sacc) + (K - _lsum(cacc)).astype(jnp.float32) * t
    res_v[...] = sgn * ssum / jnp.float32(2 * K)
    pltpu.sync_copy(res_v, out_hbm.at[wid])


def _select_sc(scores):
    mesh = plsc.VectorSubcoreMesh(core_axis_name="c", subcore_axis_name="s")
    fn = functools.partial(
        pl.kernel,
        mesh=mesh,
        compiler_params=pltpu.CompilerParams(needs_layout_passes=False),
        out_type=jax.ShapeDtypeStruct((NWORK, 16), jnp.float32),
        scratch_types=[
            pltpu.VMEM((N,), jnp.float32),
            pltpu.VMEM((HB,), jnp.int32),
            pltpu.VMEM((256,), jnp.int32),
            pltpu.VMEM((16,), jnp.float32),
            pltpu.VMEM((16,), jnp.int32),
        ],
    )(_select_body)
    return fn(scores)


def kernel(x, W, b):
    w_row = W.reshape(1, F)
    scores = _scores_tc(x, w_row)
    parts = _select_sc(scores)
    vals = parts[:, 0].reshape(B, 2)
    return (vals[:, 0] + vals[:, 1] + b[0]).reshape(B, 1)
